# Initial kernel scaffold; baseline (speedup 1.0000x reference)
import jax
import jax.numpy as jnp
from jax.experimental import pallas as pl


def kernel(x, edge_index, edge_weight, W1, as1, ad1, We1, ae1, b1, W2, as2, ad2, We2, ae2, b2):
    raise NotImplementedError("write your pallas kernel here")


# speedup vs baseline: 20.7861x; 20.7861x over previous
import functools

import jax
import jax.numpy as jnp
from jax import lax
from jax.experimental import pallas as pl
from jax.experimental.pallas import tpu as pltpu
from jax.experimental.pallas import tpu_sc as plsc

N = 10000
NPAD = 10240
E = 320000
D = 128
NC = 2
NS = 16
NPT = N // NS
DROWS = 640
EPT = E // (NC * NS)
BLK = 80
NBLK = EPT // BLK
RPT = NPAD // NS
ZROWS = 128

_f32 = jnp.float32


def _sc_layer_body(h_hbm, s_hbm, d_hbm, src_hbm, dst_hbm, ew_hbm, c_hbm,
                   acc_hbm, den_hbm,
                   src_v, dst_v, ew_v, sg_v, dg_v, ex_v, c_v, rows_v, sc_v,
                   z_v, zd_v, acc_sh, den_sh, sem_r, sem_s, sem_d):
    cid = lax.axis_index("c")
    sid = lax.axis_index("s")

    pltpu.sync_copy(c_hbm.at[0], c_v)
    cvec = c_v[pl.ds(0, 16)]

    zero16 = jnp.zeros((16,), _f32)

    @pl.loop(0, ZROWS)
    def _(r):
        for k in range(D // 16):
            z_v[r, pl.ds(k * 16, 16)] = zero16

    for k in range(DROWS // 16):
        zd_v[pl.ds(k * 16, 16)] = zero16

    row0 = sid * RPT
    for j in range(RPT // ZROWS):
        pltpu.sync_copy(z_v, acc_sh.at[pl.ds(row0 + j * ZROWS, ZROWS)])
    pltpu.sync_copy(zd_v, den_sh.at[pl.ds(row0, RPT)])
    plsc.subcore_barrier()

    base = (cid * NS + sid) * EPT

    @pl.loop(0, NBLK)
    def _(b):
        off = base + b * BLK
        pltpu.sync_copy(src_hbm.at[pl.ds(off, BLK)], src_v)
        pltpu.sync_copy(dst_hbm.at[pl.ds(off, BLK)], dst_v)
        pltpu.sync_copy(ew_hbm.at[pl.ds(off, BLK)], ew_v)

        cp_r = pltpu.async_copy(h_hbm.at[src_v], rows_v, sem_r)
        cp_s = pltpu.async_copy(s_hbm.at[src_v], sg_v, sem_s)
        cp_d = pltpu.async_copy(d_hbm.at[dst_v], dg_v, sem_d)
        cp_s.wait()
        cp_d.wait()

        for k in range(BLK // 16):
            sl = pl.ds(k * 16, 16)
            a = sg_v[sl] + dg_v[sl] + cvec * ew_v[sl]
            a = jnp.maximum(a, 0.2 * a)
            ex_v[sl] = jnp.exp(a)

        cp_r.wait()

        for g in range(BLK // 16):
            exg = ex_v[pl.ds(g * 16, 16)]
            for l in range(16):
                e = g * 16 + l
                exs = exg[l]
                for k in range(D // 16):
                    sl = pl.ds(k * 16, 16)
                    sc_v[e, sl] = rows_v[e, sl] * exs

        pltpu.sync_copy(sc_v, acc_sh.at[dst_v], add=True)
        pltpu.sync_copy(ex_v, den_sh.at[dst_v], add=True)

    plsc.subcore_barrier()
    pltpu.sync_copy(acc_sh.at[pl.ds(row0, RPT)],
                    acc_hbm.at[cid, pl.ds(row0, RPT)])
    pltpu.sync_copy(den_sh.at[pl.ds(row0, RPT)],
                    den_hbm.at[pl.ds(cid * NPAD + row0, RPT)])


_sc_layer = pl.kernel(
    _sc_layer_body,
    out_type=(jax.ShapeDtypeStruct((NC, NPAD, D), _f32),
              jax.ShapeDtypeStruct((NC * NPAD,), _f32)),
    mesh=plsc.VectorSubcoreMesh(core_axis_name="c", subcore_axis_name="s"),
    scratch_types=[
        pltpu.VMEM((BLK,), jnp.int32),
        pltpu.VMEM((BLK,), jnp.int32),
        pltpu.VMEM((BLK,), _f32),
        pltpu.VMEM((BLK,), _f32),
        pltpu.VMEM((BLK,), _f32),
        pltpu.VMEM((BLK,), _f32),
        pltpu.VMEM((D,), _f32),
        pltpu.VMEM((BLK, D), _f32),
        pltpu.VMEM((BLK, D), _f32),
        pltpu.VMEM((ZROWS, D), _f32),
        pltpu.VMEM((DROWS,), _f32),
        pltpu.VMEM_SHARED((NPAD, D), _f32),
        pltpu.VMEM_SHARED((NPAD,), _f32),
        pltpu.SemaphoreType.DMA,
        pltpu.SemaphoreType.DMA,
        pltpu.SemaphoreType.DMA,
    ],
)


def _tc_pre_body(x_ref, w_ref, asr, adr, wer, aer,
                 h_ref, s_ref, d_ref, c_ref):
    h = jnp.dot(x_ref[...], w_ref[...], preferred_element_type=_f32)
    h_ref[...] = h
    s_ref[...] = jnp.sum(h * asr[...], axis=1, keepdims=True)
    d_ref[...] = jnp.sum(h * adr[...], axis=1, keepdims=True)
    c_ref[...] = jnp.broadcast_to(
        jnp.sum(wer[...] * aer[...], axis=1, keepdims=True), (1, D))


_ROWB = 1000
_GRID = N // _ROWB


def _tc_pre(x, w, a_s, a_d, we, a_e):
    return pl.pallas_call(
        _tc_pre_body,
        grid=(_GRID,),
        in_specs=[
            pl.BlockSpec((_ROWB, D), lambda i: (i, 0)),
            pl.BlockSpec((D, D), lambda i: (0, 0)),
            pl.BlockSpec((1, D), lambda i: (0, 0)),
            pl.BlockSpec((1, D), lambda i: (0, 0)),
            pl.BlockSpec((1, D), lambda i: (0, 0)),
            pl.BlockSpec((1, D), lambda i: (0, 0)),
        ],
        out_specs=[
            pl.BlockSpec((_ROWB, D), lambda i: (i, 0)),
            pl.BlockSpec((_ROWB, 1), lambda i: (i, 0)),
            pl.BlockSpec((_ROWB, 1), lambda i: (i, 0)),
            pl.BlockSpec((1, D), lambda i: (0, 0)),
        ],
        out_shape=[
            jax.ShapeDtypeStruct((N, D), _f32),
            jax.ShapeDtypeStruct((N, 1), _f32),
            jax.ShapeDtypeStruct((N, 1), _f32),
            jax.ShapeDtypeStruct((1, D), _f32),
        ],
    )(x, w, a_s, a_d, we, a_e)


def _combine(acc_blk, den_blk, b_row):
    u = acc_blk[0] + acc_blk[1]
    den = den_blk[0] + den_blk[1]
    return u / (den + 1e-16) + b_row


def _tc_mid_body(acc_ref, den_ref, b1r, w_ref, asr, adr, wer, aer,
                 h_ref, s_ref, d_ref, c_ref):
    x1 = _combine(acc_ref[...], den_ref[...], b1r[...])
    x2 = jnp.where(x1 > 0, x1, jnp.exp(x1) - 1.0)
    h = jnp.dot(x2, w_ref[...], preferred_element_type=_f32)
    h_ref[...] = h
    s_ref[...] = jnp.sum(h * asr[...], axis=1, keepdims=True)
    d_ref[...] = jnp.sum(h * adr[...], axis=1, keepdims=True)
    c_ref[...] = jnp.broadcast_to(
        jnp.sum(wer[...] * aer[...], axis=1, keepdims=True), (1, D))


def _tc_mid(acc, den, b1, w, a_s, a_d, we, a_e):
    return pl.pallas_call(
        _tc_mid_body,
        grid=(_GRID,),
        in_specs=[
            pl.BlockSpec((NC, _ROWB, D), lambda i: (0, i, 0)),
            pl.BlockSpec((NC, _ROWB, 1), lambda i: (0, i, 0)),
            pl.BlockSpec((1, D), lambda i: (0, 0)),
            pl.BlockSpec((D, D), lambda i: (0, 0)),
            pl.BlockSpec((1, D), lambda i: (0, 0)),
            pl.BlockSpec((1, D), lambda i: (0, 0)),
            pl.BlockSpec((1, D), lambda i: (0, 0)),
            pl.BlockSpec((1, D), lambda i: (0, 0)),
        ],
        out_specs=[
            pl.BlockSpec((_ROWB, D), lambda i: (i, 0)),
            pl.BlockSpec((_ROWB, 1), lambda i: (i, 0)),
            pl.BlockSpec((_ROWB, 1), lambda i: (i, 0)),
            pl.BlockSpec((1, D), lambda i: (0, 0)),
        ],
        out_shape=[
            jax.ShapeDtypeStruct((N, D), _f32),
            jax.ShapeDtypeStruct((N, 1), _f32),
            jax.ShapeDtypeStruct((N, 1), _f32),
            jax.ShapeDtypeStruct((1, D), _f32),
        ],
    )(acc, den, b1, w, a_s, a_d, we, a_e)


def _tc_post_body(acc_ref, den_ref, b2r, o_ref):
    o_ref[...] = _combine(acc_ref[...], den_ref[...], b2r[...])


def _tc_post(acc, den, b2):
    return pl.pallas_call(
        _tc_post_body,
        grid=(_GRID,),
        in_specs=[
            pl.BlockSpec((NC, _ROWB, D), lambda i: (0, i, 0)),
            pl.BlockSpec((NC, _ROWB, 1), lambda i: (0, i, 0)),
            pl.BlockSpec((1, D), lambda i: (0, 0)),
        ],
        out_specs=pl.BlockSpec((_ROWB, D), lambda i: (i, 0)),
        out_shape=jax.ShapeDtypeStruct((N, D), _f32),
    )(acc, den, b2)


@jax.jit
def kernel(x, edge_index, edge_weight, W1, as1, ad1, We1, ae1, b1,
           W2, as2, ad2, We2, ae2, b2):
    src = edge_index[0]
    dst = edge_index[1]
    ew = edge_weight.reshape(E)

    h1, s1, d1, c1 = _tc_pre(x, W1, as1, ad1, We1, ae1)
    acc1, den1 = _sc_layer(h1, s1.reshape(N), d1.reshape(N), src, dst, ew, c1)
    h2, s2, d2, c2 = _tc_mid(acc1[:, :N],
                             den1.reshape(NC, NPAD, 1)[:, :N],
                             b1.reshape(1, D),
                             W2, as2, ad2, We2, ae2)
    acc2, den2 = _sc_layer(h2, s2.reshape(N), d2.reshape(N), src, dst, ew, c2)
    return _tc_post(acc2[:, :N], den2.reshape(NC, NPAD, 1)[:, :N],
                    b2.reshape(1, D))


# speedup vs baseline: 30.7336x; 1.4786x over previous
import functools

import jax
import jax.numpy as jnp
from jax import lax
from jax.experimental import pallas as pl
from jax.experimental.pallas import tpu as pltpu
from jax.experimental.pallas import tpu_sc as plsc

N = 10000
NPAD = 10240
E = 320000
D = 128
NC = 2
NS = 16
NPT = N // NS
DROWS = 640
EPT = E // (NC * NS)
BLK = 80
NBLK = EPT // BLK
RPT = NPAD // NS
ZROWS = 128

_f32 = jnp.float32


def _sc_layer_body(h_hbm, s_hbm, d_hbm, src_hbm, dst_hbm, ew_hbm, c_hbm,
                   acc_hbm, den_hbm,
                   src0, src1, dst0, dst1, ew0, ew1, sg0, sg1, dg0, dg1,
                   rows0, rows1, ex_v, c_v, z_v, zd_v, acc_sh, den_sh,
                   sem_i0, sem_i1, sem_r0, sem_r1, sem_s0, sem_s1,
                   sem_d0, sem_d1):
    cid = lax.axis_index("c")
    sid = lax.axis_index("s")

    srcs = (src0, src1)
    dsts = (dst0, dst1)
    ews = (ew0, ew1)
    sgs = (sg0, sg1)
    dgs = (dg0, dg1)
    rows = (rows0, rows1)
    sem_i = (sem_i0, sem_i1)
    sem_r = (sem_r0, sem_r1)
    sem_s = (sem_s0, sem_s1)
    sem_d = (sem_d0, sem_d1)

    pltpu.sync_copy(c_hbm.at[0], c_v)
    cvec = c_v[pl.ds(0, 16)]

    zero16 = jnp.zeros((16,), _f32)

    @pl.loop(0, ZROWS)
    def _(r):
        for k in range(D // 16):
            z_v[r, pl.ds(k * 16, 16)] = zero16

    for k in range(DROWS // 16):
        zd_v[pl.ds(k * 16, 16)] = zero16

    row0 = sid * RPT
    for j in range(RPT // ZROWS):
        pltpu.sync_copy(z_v, acc_sh.at[pl.ds(row0 + j * ZROWS, ZROWS)])
    pltpu.sync_copy(zd_v, den_sh.at[pl.ds(row0, RPT)])
    plsc.subcore_barrier()

    base = (cid * NS + sid) * EPT

    def start_idx(off, u):
        pltpu.async_copy(src_hbm.at[pl.ds(off, BLK)], srcs[u], sem_i[u])
        pltpu.async_copy(dst_hbm.at[pl.ds(off, BLK)], dsts[u], sem_i[u])
        pltpu.async_copy(ew_hbm.at[pl.ds(off, BLK)], ews[u], sem_i[u])

    def wait_idx(u):
        pltpu.make_async_copy(src_hbm.at[pl.ds(0, BLK)], srcs[u],
                              sem_i[u]).wait()
        pltpu.make_async_copy(dst_hbm.at[pl.ds(0, BLK)], dsts[u],
                              sem_i[u]).wait()
        pltpu.make_async_copy(ew_hbm.at[pl.ds(0, BLK)], ews[u],
                              sem_i[u]).wait()

    def start_gathers(u):
        pltpu.async_copy(h_hbm.at[srcs[u]], rows[u], sem_r[u])
        pltpu.async_copy(s_hbm.at[srcs[u]], sgs[u], sem_s[u])
        pltpu.async_copy(d_hbm.at[dsts[u]], dgs[u], sem_d[u])

    def wait_gathers(u):
        pltpu.make_async_copy(h_hbm.at[srcs[u]], rows[u], sem_r[u]).wait()
        pltpu.make_async_copy(s_hbm.at[srcs[u]], sgs[u], sem_s[u]).wait()
        pltpu.make_async_copy(d_hbm.at[dsts[u]], dgs[u], sem_d[u]).wait()

    def compute_scatter(u):
        for k in range(BLK // 16):
            sl = pl.ds(k * 16, 16)
            a = sgs[u][sl] + dgs[u][sl] + cvec * ews[u][sl]
            a = jnp.maximum(a, 0.2 * a)
            ex_v[sl] = jnp.exp(a)

        for g in range(BLK // 16):
            exg = ex_v[pl.ds(g * 16, 16)]
            for l in range(16):
                e = g * 16 + l
                exs = exg[l]
                for k in range(D // 16):
                    sl = pl.ds(k * 16, 16)
                    rows[u][e, sl] = rows[u][e, sl] * exs
        pltpu.sync_copy(rows[u], acc_sh.at[dsts[u]], add=True)
        pltpu.sync_copy(ex_v, den_sh.at[dsts[u]], add=True)

    start_idx(base, 0)
    start_idx(base + BLK, 1)
    wait_idx(0)
    start_gathers(0)

    @pl.loop(0, NBLK // 2)
    def _(j):
        for u in (0, 1):
            v = 1 - u
            i = 2 * j + u
            wait_idx(v)
            start_gathers(v)
            wait_gathers(u)
            compute_scatter(u)
            rel = jnp.minimum(i + 2, NBLK - 1)
            start_idx(base + rel * BLK, u)

    wait_gathers(0)
    compute_scatter(0)
    wait_idx(1)

    plsc.subcore_barrier()
    pltpu.sync_copy(acc_sh.at[pl.ds(row0, RPT)],
                    acc_hbm.at[cid, pl.ds(row0, RPT)])
    pltpu.sync_copy(den_sh.at[pl.ds(row0, RPT)],
                    den_hbm.at[pl.ds(cid * NPAD + row0, RPT)])


_sc_layer = pl.kernel(
    _sc_layer_body,
    out_type=(jax.ShapeDtypeStruct((NC, NPAD, D), _f32),
              jax.ShapeDtypeStruct((NC * NPAD,), _f32)),
    mesh=plsc.VectorSubcoreMesh(core_axis_name="c", subcore_axis_name="s"),
    scratch_types=[
        pltpu.VMEM((BLK,), jnp.int32),
        pltpu.VMEM((BLK,), jnp.int32),
        pltpu.VMEM((BLK,), jnp.int32),
        pltpu.VMEM((BLK,), jnp.int32),
        pltpu.VMEM((BLK,), _f32),
        pltpu.VMEM((BLK,), _f32),
        pltpu.VMEM((BLK,), _f32),
        pltpu.VMEM((BLK,), _f32),
        pltpu.VMEM((BLK,), _f32),
        pltpu.VMEM((BLK,), _f32),
        pltpu.VMEM((BLK, D), _f32),
        pltpu.VMEM((BLK, D), _f32),
        pltpu.VMEM((BLK,), _f32),
        pltpu.VMEM((D,), _f32),
        pltpu.VMEM((ZROWS, D), _f32),
        pltpu.VMEM((DROWS,), _f32),
        pltpu.VMEM_SHARED((NPAD, D), _f32),
        pltpu.VMEM_SHARED((NPAD,), _f32),
        pltpu.SemaphoreType.DMA,
        pltpu.SemaphoreType.DMA,
        pltpu.SemaphoreType.DMA,
        pltpu.SemaphoreType.DMA,
        pltpu.SemaphoreType.DMA,
        pltpu.SemaphoreType.DMA,
        pltpu.SemaphoreType.DMA,
        pltpu.SemaphoreType.DMA,
    ],
)


def _tc_pre_body(x_ref, w_ref, asr, adr, wer, aer,
                 h_ref, s_ref, d_ref, c_ref):
    h = jnp.dot(x_ref[...], w_ref[...], preferred_element_type=_f32)
    h_ref[...] = h
    s_ref[...] = jnp.sum(h * asr[...], axis=1, keepdims=True)
    d_ref[...] = jnp.sum(h * adr[...], axis=1, keepdims=True)
    c_ref[...] = jnp.broadcast_to(
        jnp.sum(wer[...] * aer[...], axis=1, keepdims=True), (1, D))


_ROWB = 1000
_GRID = N // _ROWB


def _tc_pre(x, w, a_s, a_d, we, a_e):
    return pl.pallas_call(
        _tc_pre_body,
        grid=(_GRID,),
        in_specs=[
            pl.BlockSpec((_ROWB, D), lambda i: (i, 0)),
            pl.BlockSpec((D, D), lambda i: (0, 0)),
            pl.BlockSpec((1, D), lambda i: (0, 0)),
            pl.BlockSpec((1, D), lambda i: (0, 0)),
            pl.BlockSpec((1, D), lambda i: (0, 0)),
            pl.BlockSpec((1, D), lambda i: (0, 0)),
        ],
        out_specs=[
            pl.BlockSpec((_ROWB, D), lambda i: (i, 0)),
            pl.BlockSpec((_ROWB, 1), lambda i: (i, 0)),
            pl.BlockSpec((_ROWB, 1), lambda i: (i, 0)),
            pl.BlockSpec((1, D), lambda i: (0, 0)),
        ],
        out_shape=[
            jax.ShapeDtypeStruct((N, D), _f32),
            jax.ShapeDtypeStruct((N, 1), _f32),
            jax.ShapeDtypeStruct((N, 1), _f32),
            jax.ShapeDtypeStruct((1, D), _f32),
        ],
    )(x, w, a_s, a_d, we, a_e)


def _combine(acc_blk, den_blk, b_row):
    u = acc_blk[0] + acc_blk[1]
    den = den_blk[0] + den_blk[1]
    return u / (den + 1e-16) + b_row


def _tc_mid_body(acc_ref, den_ref, b1r, w_ref, asr, adr, wer, aer,
                 h_ref, s_ref, d_ref, c_ref):
    x1 = _combine(acc_ref[...], den_ref[...], b1r[...])
    x2 = jnp.where(x1 > 0, x1, jnp.exp(x1) - 1.0)
    h = jnp.dot(x2, w_ref[...], preferred_element_type=_f32)
    h_ref[...] = h
    s_ref[...] = jnp.sum(h * asr[...], axis=1, keepdims=True)
    d_ref[...] = jnp.sum(h * adr[...], axis=1, keepdims=True)
    c_ref[...] = jnp.broadcast_to(
        jnp.sum(wer[...] * aer[...], axis=1, keepdims=True), (1, D))


def _tc_mid(acc, den, b1, w, a_s, a_d, we, a_e):
    return pl.pallas_call(
        _tc_mid_body,
        grid=(_GRID,),
        in_specs=[
            pl.BlockSpec((NC, _ROWB, D), lambda i: (0, i, 0)),
            pl.BlockSpec((NC, _ROWB, 1), lambda i: (0, i, 0)),
            pl.BlockSpec((1, D), lambda i: (0, 0)),
            pl.BlockSpec((D, D), lambda i: (0, 0)),
            pl.BlockSpec((1, D), lambda i: (0, 0)),
            pl.BlockSpec((1, D), lambda i: (0, 0)),
            pl.BlockSpec((1, D), lambda i: (0, 0)),
            pl.BlockSpec((1, D), lambda i: (0, 0)),
        ],
        out_specs=[
            pl.BlockSpec((_ROWB, D), lambda i: (i, 0)),
            pl.BlockSpec((_ROWB, 1), lambda i: (i, 0)),
            pl.BlockSpec((_ROWB, 1), lambda i: (i, 0)),
            pl.BlockSpec((1, D), lambda i: (0, 0)),
        ],
        out_shape=[
            jax.ShapeDtypeStruct((N, D), _f32),
            jax.ShapeDtypeStruct((N, 1), _f32),
            jax.ShapeDtypeStruct((N, 1), _f32),
            jax.ShapeDtypeStruct((1, D), _f32),
        ],
    )(acc, den, b1, w, a_s, a_d, we, a_e)


def _tc_post_body(acc_ref, den_ref, b2r, o_ref):
    o_ref[...] = _combine(acc_ref[...], den_ref[...], b2r[...])


def _tc_post(acc, den, b2):
    return pl.pallas_call(
        _tc_post_body,
        grid=(_GRID,),
        in_specs=[
            pl.BlockSpec((NC, _ROWB, D), lambda i: (0, i, 0)),
            pl.BlockSpec((NC, _ROWB, 1), lambda i: (0, i, 0)),
            pl.BlockSpec((1, D), lambda i: (0, 0)),
        ],
        out_specs=pl.BlockSpec((_ROWB, D), lambda i: (i, 0)),
        out_shape=jax.ShapeDtypeStruct((N, D), _f32),
    )(acc, den, b2)


@jax.jit
def kernel(x, edge_index, edge_weight, W1, as1, ad1, We1, ae1, b1,
           W2, as2, ad2, We2, ae2, b2):
    src = edge_index[0]
    dst = edge_index[1]
    ew = edge_weight.reshape(E)

    h1, s1, d1, c1 = _tc_pre(x, W1, as1, ad1, We1, ae1)
    acc1, den1 = _sc_layer(h1, s1.reshape(N), d1.reshape(N), src, dst, ew, c1)
    h2, s2, d2, c2 = _tc_mid(acc1[:, :N],
                             den1.reshape(NC, NPAD, 1)[:, :N],
                             b1.reshape(1, D),
                             W2, as2, ad2, We2, ae2)
    acc2, den2 = _sc_layer(h2, s2.reshape(N), d2.reshape(N), src, dst, ew, c2)
    return _tc_post(acc2[:, :N], den2.reshape(NC, NPAD, 1)[:, :N],
                    b2.reshape(1, D))


# speedup vs baseline: 31.9402x; 1.0393x over previous
import functools

import jax
import jax.numpy as jnp
from jax import lax
from jax.experimental import pallas as pl
from jax.experimental.pallas import tpu as pltpu
from jax.experimental.pallas import tpu_sc as plsc

N = 10000
NPAD = 10240
E = 320000
D = 128
NC = 2
NS = 16
NPT = N // NS
DROWS = 640
EPT = E // (NC * NS)
BLK = 80
NBLK = EPT // BLK
RPT = NPAD // NS
ZROWS = 128

_f32 = jnp.float32


def _sc_layer_body(h_hbm, s_hbm, d_hbm, src_hbm, dst_hbm, ew_hbm, c_hbm,
                   acc_hbm, den_hbm,
                   src0, src1, dst0, dst1, ew0, ew1, sg0, sg1, dg0, dg1,
                   rows0, rows1, ex_v, c_v, z_v, zd_v, acc_sh, den_sh,
                   sem_i0, sem_i1, sem_r0, sem_r1, sem_s0, sem_s1,
                   sem_d0, sem_d1):
    cid = lax.axis_index("c")
    sid = lax.axis_index("s")

    srcs = (src0, src1)
    dsts = (dst0, dst1)
    ews = (ew0, ew1)
    sgs = (sg0, sg1)
    dgs = (dg0, dg1)
    rows = (rows0, rows1)
    sem_i = (sem_i0, sem_i1)
    sem_r = (sem_r0, sem_r1)
    sem_s = (sem_s0, sem_s1)
    sem_d = (sem_d0, sem_d1)

    pltpu.sync_copy(c_hbm.at[0], c_v)
    cvec = c_v[pl.ds(0, 16)]

    zero16 = jnp.zeros((16,), _f32)

    @pl.loop(0, ZROWS)
    def _(r):
        for k in range(D // 16):
            z_v[r, pl.ds(k * 16, 16)] = zero16

    for k in range(DROWS // 16):
        zd_v[pl.ds(k * 16, 16)] = zero16

    row0 = sid * RPT
    for j in range(RPT // ZROWS):
        pltpu.sync_copy(z_v, acc_sh.at[pl.ds(row0 + j * ZROWS, ZROWS)])
    pltpu.sync_copy(zd_v, den_sh.at[pl.ds(row0, RPT)])
    plsc.subcore_barrier()

    base = (cid * NS + sid) * EPT

    def start_idx(off, u):
        pltpu.async_copy(src_hbm.at[pl.ds(off, BLK)], srcs[u], sem_i[u])
        pltpu.async_copy(dst_hbm.at[pl.ds(off, BLK)], dsts[u], sem_i[u])
        pltpu.async_copy(ew_hbm.at[pl.ds(off, BLK)], ews[u], sem_i[u])

    def wait_idx(u):
        pltpu.make_async_copy(src_hbm.at[pl.ds(0, BLK)], srcs[u],
                              sem_i[u]).wait()
        pltpu.make_async_copy(dst_hbm.at[pl.ds(0, BLK)], dsts[u],
                              sem_i[u]).wait()
        pltpu.make_async_copy(ew_hbm.at[pl.ds(0, BLK)], ews[u],
                              sem_i[u]).wait()

    def start_gathers(u):
        pltpu.async_copy(h_hbm.at[srcs[u]], rows[u], sem_r[u])
        pltpu.async_copy(s_hbm.at[srcs[u]], sgs[u], sem_s[u])
        pltpu.async_copy(d_hbm.at[dsts[u]], dgs[u], sem_d[u])

    def wait_gathers(u):
        pltpu.make_async_copy(h_hbm.at[srcs[u]], rows[u], sem_r[u]).wait()
        pltpu.make_async_copy(s_hbm.at[srcs[u]], sgs[u], sem_s[u]).wait()
        pltpu.make_async_copy(d_hbm.at[dsts[u]], dgs[u], sem_d[u]).wait()

    def compute_scatter(u):
        for k in range(BLK // 16):
            sl = pl.ds(k * 16, 16)
            a = sgs[u][sl] + dgs[u][sl] + cvec * ews[u][sl]
            a = jnp.maximum(a, 0.2 * a)
            ex_v[sl] = jnp.exp(a)

        for g in range(BLK // 16):
            exg = ex_v[pl.ds(g * 16, 16)]
            for l in range(16):
                e = g * 16 + l
                exs = exg[l]
                for k in range(D // 16):
                    sl = pl.ds(k * 16, 16)
                    rows[u][e, sl] = rows[u][e, sl] * exs
        pltpu.sync_copy(rows[u], acc_sh.at[dsts[u]], add=True)
        pltpu.sync_copy(ex_v, den_sh.at[dsts[u]], add=True)

    start_idx(base, 0)
    start_idx(base + BLK, 1)
    wait_idx(0)
    start_gathers(0)

    @pl.loop(0, NBLK // 2)
    def _(j):
        for u in (0, 1):
            v = 1 - u
            i = 2 * j + u
            wait_idx(v)
            start_gathers(v)
            wait_gathers(u)
            compute_scatter(u)
            rel = jnp.minimum(i + 2, NBLK - 1)
            start_idx(base + rel * BLK, u)

    wait_gathers(0)
    compute_scatter(0)
    wait_idx(1)

    plsc.subcore_barrier()
    pltpu.sync_copy(acc_sh.at[pl.ds(row0, RPT)],
                    acc_hbm.at[cid, pl.ds(row0, RPT)])
    pltpu.sync_copy(den_sh.at[pl.ds(row0, RPT)],
                    den_hbm.at[pl.ds(cid * NPAD + row0, RPT)])


_sc_layer = pl.kernel(
    _sc_layer_body,
    out_type=(jax.ShapeDtypeStruct((NC, NPAD, D), _f32),
              jax.ShapeDtypeStruct((NC * NPAD,), _f32)),
    mesh=plsc.VectorSubcoreMesh(core_axis_name="c", subcore_axis_name="s"),
    scratch_types=[
        pltpu.VMEM((BLK,), jnp.int32),
        pltpu.VMEM((BLK,), jnp.int32),
        pltpu.VMEM((BLK,), jnp.int32),
        pltpu.VMEM((BLK,), jnp.int32),
        pltpu.VMEM((BLK,), _f32),
        pltpu.VMEM((BLK,), _f32),
        pltpu.VMEM((BLK,), _f32),
        pltpu.VMEM((BLK,), _f32),
        pltpu.VMEM((BLK,), _f32),
        pltpu.VMEM((BLK,), _f32),
        pltpu.VMEM((BLK, D), _f32),
        pltpu.VMEM((BLK, D), _f32),
        pltpu.VMEM((BLK,), _f32),
        pltpu.VMEM((D,), _f32),
        pltpu.VMEM((ZROWS, D), _f32),
        pltpu.VMEM((DROWS,), _f32),
        pltpu.VMEM_SHARED((NPAD, D), _f32),
        pltpu.VMEM_SHARED((NPAD,), _f32),
        pltpu.SemaphoreType.DMA,
        pltpu.SemaphoreType.DMA,
        pltpu.SemaphoreType.DMA,
        pltpu.SemaphoreType.DMA,
        pltpu.SemaphoreType.DMA,
        pltpu.SemaphoreType.DMA,
        pltpu.SemaphoreType.DMA,
        pltpu.SemaphoreType.DMA,
    ],
)


def _tc_pre_body(x_ref, w_ref, asr, adr, wer, aer,
                 h_ref, s_ref, d_ref, c_ref):
    h = jnp.dot(x_ref[...], w_ref[...], preferred_element_type=_f32)
    h_ref[...] = h
    s_ref[...] = jnp.sum(h * asr[...], axis=1, keepdims=True)
    d_ref[...] = jnp.sum(h * adr[...], axis=1, keepdims=True)
    c_ref[...] = jnp.broadcast_to(
        jnp.sum(wer[...] * aer[...], axis=1, keepdims=True), (1, D))


_ROWB = 1000
_GRID = N // _ROWB


def _tc_pre(x, w, a_s, a_d, we, a_e):
    return pl.pallas_call(
        _tc_pre_body,
        grid=(_GRID,),
        in_specs=[
            pl.BlockSpec((_ROWB, D), lambda i: (i, 0)),
            pl.BlockSpec((D, D), lambda i: (0, 0)),
            pl.BlockSpec((1, D), lambda i: (0, 0)),
            pl.BlockSpec((1, D), lambda i: (0, 0)),
            pl.BlockSpec((1, D), lambda i: (0, 0)),
            pl.BlockSpec((1, D), lambda i: (0, 0)),
        ],
        out_specs=[
            pl.BlockSpec((_ROWB, D), lambda i: (i, 0)),
            pl.BlockSpec((_ROWB, 1), lambda i: (i, 0)),
            pl.BlockSpec((_ROWB, 1), lambda i: (i, 0)),
            pl.BlockSpec((1, D), lambda i: (0, 0)),
        ],
        out_shape=[
            jax.ShapeDtypeStruct((N, D), _f32),
            jax.ShapeDtypeStruct((N, 1), _f32),
            jax.ShapeDtypeStruct((N, 1), _f32),
            jax.ShapeDtypeStruct((1, D), _f32),
        ],
    )(x, w, a_s, a_d, we, a_e)


def _combine(acc_blk, den_blk, b_row):
    u = acc_blk[0] + acc_blk[1]
    den = den_blk[0] + den_blk[1]
    return u / (den + 1e-16) + b_row


def _tc_mid_body(acc_ref, den_ref, b1r, w_ref, asr, adr, wer, aer,
                 h_ref, s_ref, d_ref, c_ref):
    x1 = _combine(acc_ref[...], den_ref[...], b1r[...])
    x2 = jnp.where(x1 > 0, x1, jnp.exp(x1) - 1.0)
    h = jnp.dot(x2, w_ref[...], preferred_element_type=_f32)
    h_ref[...] = h
    s_ref[...] = jnp.sum(h * asr[...], axis=1, keepdims=True)
    d_ref[...] = jnp.sum(h * adr[...], axis=1, keepdims=True)
    c_ref[...] = jnp.broadcast_to(
        jnp.sum(wer[...] * aer[...], axis=1, keepdims=True), (1, D))


def _tc_mid(acc, den, b1, w, a_s, a_d, we, a_e):
    return pl.pallas_call(
        _tc_mid_body,
        grid=(_GRID,),
        in_specs=[
            pl.BlockSpec((NC, _ROWB, D), lambda i: (0, i, 0)),
            pl.BlockSpec((NC, _ROWB, 1), lambda i: (0, i, 0)),
            pl.BlockSpec((1, D), lambda i: (0, 0)),
            pl.BlockSpec((D, D), lambda i: (0, 0)),
            pl.BlockSpec((1, D), lambda i: (0, 0)),
            pl.BlockSpec((1, D), lambda i: (0, 0)),
            pl.BlockSpec((1, D), lambda i: (0, 0)),
            pl.BlockSpec((1, D), lambda i: (0, 0)),
        ],
        out_specs=[
            pl.BlockSpec((_ROWB, D), lambda i: (i, 0)),
            pl.BlockSpec((_ROWB, 1), lambda i: (i, 0)),
            pl.BlockSpec((_ROWB, 1), lambda i: (i, 0)),
            pl.BlockSpec((1, D), lambda i: (0, 0)),
        ],
        out_shape=[
            jax.ShapeDtypeStruct((N, D), _f32),
            jax.ShapeDtypeStruct((N, 1), _f32),
            jax.ShapeDtypeStruct((N, 1), _f32),
            jax.ShapeDtypeStruct((1, D), _f32),
        ],
    )(acc, den, b1, w, a_s, a_d, we, a_e)


def _tc_post_body(acc_ref, den_ref, b2r, o_ref):
    o_ref[...] = _combine(acc_ref[...], den_ref[...], b2r[...])


def _tc_post(acc, den, b2):
    return pl.pallas_call(
        _tc_post_body,
        grid=(_GRID,),
        in_specs=[
            pl.BlockSpec((NC, _ROWB, D), lambda i: (0, i, 0)),
            pl.BlockSpec((NC, _ROWB, 1), lambda i: (0, i, 0)),
            pl.BlockSpec((1, D), lambda i: (0, 0)),
        ],
        out_specs=pl.BlockSpec((_ROWB, D), lambda i: (i, 0)),
        out_shape=jax.ShapeDtypeStruct((N, D), _f32),
    )(acc, den, b2)


@jax.jit
def kernel(x, edge_index, edge_weight, W1, as1, ad1, We1, ae1, b1,
           W2, as2, ad2, We2, ae2, b2):
    src = edge_index[0]
    dst = edge_index[1]
    ew = edge_weight.reshape(E)

    h1, s1, d1, c1 = _tc_pre(x, W1, as1, ad1, We1, ae1)
    acc1, den1 = _sc_layer(h1, s1.reshape(N), d1.reshape(N), src, dst, ew, c1)
    h2, s2, d2, c2 = _tc_mid(acc1, den1.reshape(NC, NPAD, 1),
                             b1.reshape(1, D),
                             W2, as2, ad2, We2, ae2)
    acc2, den2 = _sc_layer(h2, s2.reshape(N), d2.reshape(N), src, dst, ew, c2)
    return _tc_post(acc2, den2.reshape(NC, NPAD, 1), b2.reshape(1, D))


# speedup vs baseline: 36.3115x; 1.1369x over previous
import functools

import jax
import jax.numpy as jnp
from jax import lax
from jax.experimental import pallas as pl
from jax.experimental.pallas import tpu as pltpu
from jax.experimental.pallas import tpu_sc as plsc

N = 10000
NPAD = 10240
E = 320000
D = 128
NC = 2
NS = 16
NPT = N // NS
DROWS = 640
EPT = E // (NC * NS)
BLK = 80
NBLK = EPT // BLK
RPT = NPAD // NS
ZROWS = 64

_f32 = jnp.float32


def _sc_layer_body(h_hbm, s_hbm, d_hbm, src_hbm, dst_hbm, ew_hbm, c_hbm,
                   acc_hbm, den_hbm,
                   src0, src1, src2, dst0, dst1, dst2, ew0, ew1, ew2,
                   sg0, sg1, sg2, dg0, dg1, dg2, rows0, rows1, rows2,
                   ex0, ex1, ex2, c_v, z_v, zd_v, acc_sh, den_sh,
                   sem_i0, sem_i1, sem_i2, sem_r0, sem_r1, sem_r2,
                   sem_s0, sem_s1, sem_s2, sem_d0, sem_d1, sem_d2,
                   sem_a0, sem_a1, sem_a2, sem_n0, sem_n1, sem_n2):
    cid = lax.axis_index("c")
    sid = lax.axis_index("s")

    srcs = (src0, src1, src2)
    dsts = (dst0, dst1, dst2)
    ews = (ew0, ew1, ew2)
    sgs = (sg0, sg1, sg2)
    dgs = (dg0, dg1, dg2)
    rows = (rows0, rows1, rows2)
    exs_v = (ex0, ex1, ex2)
    sem_i = (sem_i0, sem_i1, sem_i2)
    sem_r = (sem_r0, sem_r1, sem_r2)
    sem_s = (sem_s0, sem_s1, sem_s2)
    sem_d = (sem_d0, sem_d1, sem_d2)
    sem_a = (sem_a0, sem_a1, sem_a2)
    sem_n = (sem_n0, sem_n1, sem_n2)

    pltpu.sync_copy(c_hbm.at[0], c_v)
    cvec = c_v[pl.ds(0, 16)]

    zero16 = jnp.zeros((16,), _f32)

    @pl.loop(0, ZROWS)
    def _(r):
        for k in range(D // 16):
            z_v[r, pl.ds(k * 16, 16)] = zero16

    for k in range(DROWS // 16):
        zd_v[pl.ds(k * 16, 16)] = zero16

    row0 = sid * RPT
    for j in range(RPT // ZROWS):
        pltpu.sync_copy(z_v, acc_sh.at[pl.ds(row0 + j * ZROWS, ZROWS)])
    pltpu.sync_copy(zd_v, den_sh.at[pl.ds(row0, RPT)])
    plsc.subcore_barrier()

    base = (cid * NS + sid) * EPT

    def start_idx(off, u):
        pltpu.async_copy(src_hbm.at[pl.ds(off, BLK)], srcs[u], sem_i[u])
        pltpu.async_copy(dst_hbm.at[pl.ds(off, BLK)], dsts[u], sem_i[u])
        pltpu.async_copy(ew_hbm.at[pl.ds(off, BLK)], ews[u], sem_i[u])

    def wait_idx(u):
        pltpu.make_async_copy(src_hbm.at[pl.ds(0, BLK)], srcs[u],
                              sem_i[u]).wait()
        pltpu.make_async_copy(dst_hbm.at[pl.ds(0, BLK)], dsts[u],
                              sem_i[u]).wait()
        pltpu.make_async_copy(ew_hbm.at[pl.ds(0, BLK)], ews[u],
                              sem_i[u]).wait()

    def start_gathers(u):
        pltpu.async_copy(h_hbm.at[srcs[u]], rows[u], sem_r[u])
        pltpu.async_copy(s_hbm.at[srcs[u]], sgs[u], sem_s[u])
        pltpu.async_copy(d_hbm.at[dsts[u]], dgs[u], sem_d[u])

    def wait_gathers(u):
        pltpu.make_async_copy(h_hbm.at[srcs[u]], rows[u], sem_r[u]).wait()
        pltpu.make_async_copy(s_hbm.at[srcs[u]], sgs[u], sem_s[u]).wait()
        pltpu.make_async_copy(d_hbm.at[dsts[u]], dgs[u], sem_d[u]).wait()

    def compute(u):
        for k in range(BLK // 16):
            sl = pl.ds(k * 16, 16)
            a = sgs[u][sl] + dgs[u][sl] + cvec * ews[u][sl]
            a = jnp.maximum(a, 0.2 * a)
            exs_v[u][sl] = jnp.exp(a)

        for g in range(BLK // 16):
            exg = exs_v[u][pl.ds(g * 16, 16)]
            for l in range(16):
                e = g * 16 + l
                exs = exg[l]
                for k in range(D // 16):
                    sl = pl.ds(k * 16, 16)
                    rows[u][e, sl] = rows[u][e, sl] * exs

    def start_scatter(u):
        pltpu.async_copy(rows[u], acc_sh.at[dsts[u]], sem_a[u], add=True)
        pltpu.async_copy(exs_v[u], den_sh.at[dsts[u]], sem_n[u], add=True)

    def wait_scatter(u):
        pltpu.make_async_copy(rows[u], acc_sh.at[dsts[u]], sem_a[u]).wait()
        pltpu.make_async_copy(exs_v[u], den_sh.at[dsts[u]], sem_n[u]).wait()

    start_idx(base, 0)
    start_idx(base + BLK, 1)
    wait_idx(0)
    start_gathers(0)

    wait_idx(1)
    start_gathers(1)
    wait_gathers(0)
    compute(0)
    start_scatter(0)
    start_idx(base + 2 * BLK, 2)

    wait_idx(2)
    start_gathers(2)
    wait_gathers(1)
    compute(1)
    start_scatter(1)
    wait_scatter(0)
    start_idx(base + 3 * BLK, 0)

    wait_idx(0)
    start_gathers(0)
    wait_gathers(2)
    compute(2)
    start_scatter(2)
    wait_scatter(1)
    start_idx(base + 4 * BLK, 1)

    @pl.loop(0, (NBLK - 5) // 3)
    def _(j):
        i0 = 3 * j + 3
        for t in range(3):
            s = t
            sn = (t + 1) % 3
            snn = (t + 2) % 3
            wait_idx(sn)
            start_gathers(sn)
            wait_gathers(s)
            compute(s)
            start_scatter(s)
            wait_scatter(snn)
            start_idx(base + (i0 + t + 2) * BLK, snn)

    wait_idx(1)
    start_gathers(1)
    wait_gathers(0)
    compute(0)
    start_scatter(0)
    wait_scatter(2)

    wait_gathers(1)
    compute(1)
    start_scatter(1)
    wait_scatter(0)
    wait_scatter(1)

    plsc.subcore_barrier()
    pltpu.sync_copy(acc_sh.at[pl.ds(row0, RPT)],
                    acc_hbm.at[cid, pl.ds(row0, RPT)])
    pltpu.sync_copy(den_sh.at[pl.ds(row0, RPT)],
                    den_hbm.at[pl.ds(cid * NPAD + row0, RPT)])


_sc_layer = pl.kernel(
    _sc_layer_body,
    out_type=(jax.ShapeDtypeStruct((NC, NPAD, D), _f32),
              jax.ShapeDtypeStruct((NC * NPAD,), _f32)),
    mesh=plsc.VectorSubcoreMesh(core_axis_name="c", subcore_axis_name="s"),
    scratch_types=(
        [pltpu.VMEM((BLK,), jnp.int32) for _ in range(6)]
        + [pltpu.VMEM((BLK,), _f32) for _ in range(9)]
        + [pltpu.VMEM((BLK, D), _f32) for _ in range(3)]
        + [pltpu.VMEM((BLK,), _f32) for _ in range(3)]
        + [
            pltpu.VMEM((D,), _f32),
            pltpu.VMEM((ZROWS, D), _f32),
            pltpu.VMEM((DROWS,), _f32),
            pltpu.VMEM_SHARED((NPAD, D), _f32),
            pltpu.VMEM_SHARED((NPAD,), _f32),
        ]
        + [pltpu.SemaphoreType.DMA for _ in range(18)]
    ),
)


def _tc_pre_body(x_ref, w_ref, asr, adr, wer, aer,
                 h_ref, s_ref, d_ref, c_ref):
    h = jnp.dot(x_ref[...], w_ref[...], preferred_element_type=_f32)
    h_ref[...] = h
    s_ref[...] = jnp.sum(h * asr[...], axis=1, keepdims=True)
    d_ref[...] = jnp.sum(h * adr[...], axis=1, keepdims=True)
    c_ref[...] = jnp.broadcast_to(
        jnp.sum(wer[...] * aer[...], axis=1, keepdims=True), (1, D))


_ROWB = 1000
_GRID = N // _ROWB


def _tc_pre(x, w, a_s, a_d, we, a_e):
    return pl.pallas_call(
        _tc_pre_body,
        grid=(_GRID,),
        in_specs=[
            pl.BlockSpec((_ROWB, D), lambda i: (i, 0)),
            pl.BlockSpec((D, D), lambda i: (0, 0)),
            pl.BlockSpec((1, D), lambda i: (0, 0)),
            pl.BlockSpec((1, D), lambda i: (0, 0)),
            pl.BlockSpec((1, D), lambda i: (0, 0)),
            pl.BlockSpec((1, D), lambda i: (0, 0)),
        ],
        out_specs=[
            pl.BlockSpec((_ROWB, D), lambda i: (i, 0)),
            pl.BlockSpec((_ROWB, 1), lambda i: (i, 0)),
            pl.BlockSpec((_ROWB, 1), lambda i: (i, 0)),
            pl.BlockSpec((1, D), lambda i: (0, 0)),
        ],
        out_shape=[
            jax.ShapeDtypeStruct((N, D), _f32),
            jax.ShapeDtypeStruct((N, 1), _f32),
            jax.ShapeDtypeStruct((N, 1), _f32),
            jax.ShapeDtypeStruct((1, D), _f32),
        ],
    )(x, w, a_s, a_d, we, a_e)


def _combine(acc_blk, den_blk, b_row):
    u = acc_blk[0] + acc_blk[1]
    den = den_blk[0] + den_blk[1]
    return u / (den + 1e-16) + b_row


def _tc_mid_body(acc_ref, den_ref, b1r, w_ref, asr, adr, wer, aer,
                 h_ref, s_ref, d_ref, c_ref):
    x1 = _combine(acc_ref[...], den_ref[...], b1r[...])
    x2 = jnp.where(x1 > 0, x1, jnp.exp(x1) - 1.0)
    h = jnp.dot(x2, w_ref[...], preferred_element_type=_f32)
    h_ref[...] = h
    s_ref[...] = jnp.sum(h * asr[...], axis=1, keepdims=True)
    d_ref[...] = jnp.sum(h * adr[...], axis=1, keepdims=True)
    c_ref[...] = jnp.broadcast_to(
        jnp.sum(wer[...] * aer[...], axis=1, keepdims=True), (1, D))


def _tc_mid(acc, den, b1, w, a_s, a_d, we, a_e):
    return pl.pallas_call(
        _tc_mid_body,
        grid=(_GRID,),
        in_specs=[
            pl.BlockSpec((NC, _ROWB, D), lambda i: (0, i, 0)),
            pl.BlockSpec((NC, _ROWB, 1), lambda i: (0, i, 0)),
            pl.BlockSpec((1, D), lambda i: (0, 0)),
            pl.BlockSpec((D, D), lambda i: (0, 0)),
            pl.BlockSpec((1, D), lambda i: (0, 0)),
            pl.BlockSpec((1, D), lambda i: (0, 0)),
            pl.BlockSpec((1, D), lambda i: (0, 0)),
            pl.BlockSpec((1, D), lambda i: (0, 0)),
        ],
        out_specs=[
            pl.BlockSpec((_ROWB, D), lambda i: (i, 0)),
            pl.BlockSpec((_ROWB, 1), lambda i: (i, 0)),
            pl.BlockSpec((_ROWB, 1), lambda i: (i, 0)),
            pl.BlockSpec((1, D), lambda i: (0, 0)),
        ],
        out_shape=[
            jax.ShapeDtypeStruct((N, D), _f32),
            jax.ShapeDtypeStruct((N, 1), _f32),
            jax.ShapeDtypeStruct((N, 1), _f32),
            jax.ShapeDtypeStruct((1, D), _f32),
        ],
    )(acc, den, b1, w, a_s, a_d, we, a_e)


def _tc_post_body(acc_ref, den_ref, b2r, o_ref):
    o_ref[...] = _combine(acc_ref[...], den_ref[...], b2r[...])


def _tc_post(acc, den, b2):
    return pl.pallas_call(
        _tc_post_body,
        grid=(_GRID,),
        in_specs=[
            pl.BlockSpec((NC, _ROWB, D), lambda i: (0, i, 0)),
            pl.BlockSpec((NC, _ROWB, 1), lambda i: (0, i, 0)),
            pl.BlockSpec((1, D), lambda i: (0, 0)),
        ],
        out_specs=pl.BlockSpec((_ROWB, D), lambda i: (i, 0)),
        out_shape=jax.ShapeDtypeStruct((N, D), _f32),
    )(acc, den, b2)


@jax.jit
def kernel(x, edge_index, edge_weight, W1, as1, ad1, We1, ae1, b1,
           W2, as2, ad2, We2, ae2, b2):
    src = edge_index[0]
    dst = edge_index[1]
    ew = edge_weight.reshape(E)

    h1, s1, d1, c1 = _tc_pre(x, W1, as1, ad1, We1, ae1)
    acc1, den1 = _sc_layer(h1, s1.reshape(N), d1.reshape(N), src, dst, ew, c1)
    h2, s2, d2, c2 = _tc_mid(acc1, den1.reshape(NC, NPAD, 1),
                             b1.reshape(1, D),
                             W2, as2, ad2, We2, ae2)
    acc2, den2 = _sc_layer(h2, s2.reshape(N), d2.reshape(N), src, dst, ew, c2)
    return _tc_post(acc2, den2.reshape(NC, NPAD, 1), b2.reshape(1, D))
